# baseline (speedup 1.0000x reference)
import jax
import jax.numpy as jnp
from jax.experimental import pallas as pl
from jax.experimental.pallas import tpu as pltpu


def _xw_body(x_ref, w1_ref, b1_ref, o_ref):
    o_ref[...] = (
        jnp.dot(x_ref[...], w1_ref[...], preferred_element_type=jnp.float32)
        + b1_ref[...]
    )


def _layer1_body(adj_ref, xw_ref, w2_ref, b2_ref, o_ref):
    h = jnp.dot(adj_ref[...], xw_ref[...], preferred_element_type=jnp.float32)
    h = jnp.maximum(h, 0.0)
    o_ref[...] = (
        jnp.dot(h, w2_ref[...], preferred_element_type=jnp.float32) + b2_ref[...]
    )


def _layer2_body(adj_ref, h2_ref, o_ref):
    o_ref[...] = jnp.dot(adj_ref[...], h2_ref[...], preferred_element_type=jnp.float32)


def kernel(Adj_, x, W1, b1, W2, b2):
    N, D = x.shape
    H = W1.shape[1]
    C = W2.shape[1]
    b1r = b1.reshape(1, H)
    b2r = b2.reshape(1, C)

    BM = 400
    BX = 1000

    xw = pl.pallas_call(
        _xw_body,
        grid=(N // BX,),
        in_specs=[
            pl.BlockSpec((BX, D), lambda m: (m, 0)),
            pl.BlockSpec((D, H), lambda m: (0, 0)),
            pl.BlockSpec((1, H), lambda m: (0, 0)),
        ],
        out_specs=pl.BlockSpec((BX, H), lambda m: (m, 0)),
        out_shape=jax.ShapeDtypeStruct((N, H), jnp.float32),
    )(x, W1, b1r)

    h2 = pl.pallas_call(
        _layer1_body,
        grid=(N // BM,),
        in_specs=[
            pl.BlockSpec((BM, N), lambda m: (m, 0)),
            pl.BlockSpec((N, H), lambda m: (0, 0)),
            pl.BlockSpec((H, C), lambda m: (0, 0)),
            pl.BlockSpec((1, C), lambda m: (0, 0)),
        ],
        out_specs=pl.BlockSpec((BM, C), lambda m: (m, 0)),
        out_shape=jax.ShapeDtypeStruct((N, C), jnp.float32),
        compiler_params=pltpu.CompilerParams(
            dimension_semantics=("arbitrary",),
        ),
    )(Adj_, xw, W2, b2r)

    out = pl.pallas_call(
        _layer2_body,
        grid=(N // BM,),
        in_specs=[
            pl.BlockSpec((BM, N), lambda m: (m, 0)),
            pl.BlockSpec((N, C), lambda m: (0, 0)),
        ],
        out_specs=pl.BlockSpec((BM, C), lambda m: (m, 0)),
        out_shape=jax.ShapeDtypeStruct((N, C), jnp.float32),
        compiler_params=pltpu.CompilerParams(
            dimension_semantics=("arbitrary",),
        ),
    )(Adj_, h2)

    return (out, Adj_)


# speedup vs baseline: 1.0303x; 1.0303x over previous
import jax
import jax.numpy as jnp
from jax.experimental import pallas as pl
from jax.experimental.pallas import tpu as pltpu


def _make_body(NM, BM):
    def body(adj_ref, x_ref, w1_ref, b1_ref, w2_ref, b2_ref, out_ref, xw_s, h2_s):
        i = pl.program_id(0)

        @pl.when(i == 0)
        def _compute_xw():
            xw_s[...] = (
                jnp.dot(x_ref[...], w1_ref[...], preferred_element_type=jnp.float32)
                + b1_ref[...]
            )

        @pl.when(jnp.logical_and(i >= 1, i <= NM))
        def _layer1():
            m = i - 1
            h = jnp.dot(adj_ref[...], xw_s[...], preferred_element_type=jnp.float32)
            h2_s[pl.ds(m * BM, BM), :] = (
                jnp.dot(
                    jnp.maximum(h, 0.0),
                    w2_ref[...],
                    preferred_element_type=jnp.float32,
                )
                + b2_ref[...]
            )

        @pl.when(i > NM)
        def _layer2():
            out_ref[...] = jnp.dot(
                adj_ref[...], h2_s[...], preferred_element_type=jnp.float32
            )

    return body


def kernel(Adj_, x, W1, b1, W2, b2):
    N, D = x.shape
    H = W1.shape[1]
    C = W2.shape[1]
    b1r = b1.reshape(1, H)
    b2r = b2.reshape(1, C)

    BM = 400
    NM = N // BM

    out = pl.pallas_call(
        _make_body(NM, BM),
        grid=(2 * NM + 1,),
        in_specs=[
            pl.BlockSpec(
                (BM, N),
                lambda i: (jnp.where(i <= NM, jnp.maximum(i - 1, 0), i - NM - 1), 0),
            ),
            pl.BlockSpec((N, D), lambda i: (0, 0)),
            pl.BlockSpec((D, H), lambda i: (0, 0)),
            pl.BlockSpec((1, H), lambda i: (0, 0)),
            pl.BlockSpec((H, C), lambda i: (0, 0)),
            pl.BlockSpec((1, C), lambda i: (0, 0)),
        ],
        out_specs=pl.BlockSpec(
            (BM, C), lambda i: (jnp.where(i <= NM, 0, i - NM - 1), 0)
        ),
        out_shape=jax.ShapeDtypeStruct((N, C), jnp.float32),
        scratch_shapes=[
            pltpu.VMEM((N, H), jnp.float32),
            pltpu.VMEM((N, C), jnp.float32),
        ],
        compiler_params=pltpu.CompilerParams(
            dimension_semantics=("arbitrary",),
        ),
    )(Adj_, x, W1, b1r, W2, b2r)

    return (out, Adj_)


# speedup vs baseline: 1.3661x; 1.3260x over previous
import jax
import jax.numpy as jnp
from jax.experimental import pallas as pl
from jax.experimental.pallas import tpu as pltpu


def _make_body(NM, BM):
    def body(
        adj_ref, x_ref, w1_ref, b1_ref, w2_ref, b2_ref,
        out_ref, adj_out_ref, xw_s, h2_s,
    ):
        i = pl.program_id(0)

        @pl.when(i == 0)
        def _compute_xw():
            xw_s[...] = (
                jnp.dot(x_ref[...], w1_ref[...], preferred_element_type=jnp.float32)
                + b1_ref[...]
            )

        @pl.when(jnp.logical_and(i >= 1, i <= NM))
        def _layer1():
            m = i - 1
            a = adj_ref[...]
            adj_out_ref[...] = a
            h = jnp.dot(a, xw_s[...], preferred_element_type=jnp.float32)
            h2_s[pl.ds(m * BM, BM), :] = (
                jnp.dot(
                    jnp.maximum(h, 0.0),
                    w2_ref[...],
                    preferred_element_type=jnp.float32,
                )
                + b2_ref[...]
            )

        @pl.when(i > NM)
        def _layer2():
            out_ref[...] = jnp.dot(
                adj_ref[...], h2_s[...], preferred_element_type=jnp.float32
            )

    return body


def kernel(Adj_, x, W1, b1, W2, b2):
    N, D = x.shape
    H = W1.shape[1]
    C = W2.shape[1]
    b1r = b1.reshape(1, H)
    b2r = b2.reshape(1, C)

    BM = 200
    NM = N // BM

    out, adj_out = pl.pallas_call(
        _make_body(NM, BM),
        grid=(2 * NM + 1,),
        in_specs=[
            pl.BlockSpec(
                (BM, N),
                lambda i: (jnp.where(i <= NM, jnp.maximum(i - 1, 0), i - NM - 1), 0),
            ),
            pl.BlockSpec((N, D), lambda i: (0, 0)),
            pl.BlockSpec((D, H), lambda i: (0, 0)),
            pl.BlockSpec((1, H), lambda i: (0, 0)),
            pl.BlockSpec((H, C), lambda i: (0, 0)),
            pl.BlockSpec((1, C), lambda i: (0, 0)),
        ],
        out_specs=[
            pl.BlockSpec(
                (BM, C), lambda i: (jnp.where(i <= NM, 0, i - NM - 1), 0)
            ),
            pl.BlockSpec(
                (BM, N),
                lambda i: (jnp.where(i <= NM, jnp.maximum(i - 1, 0), NM - 1), 0),
            ),
        ],
        out_shape=[
            jax.ShapeDtypeStruct((N, C), jnp.float32),
            jax.ShapeDtypeStruct((N, N), jnp.float32),
        ],
        scratch_shapes=[
            pltpu.VMEM((N, H), jnp.float32),
            pltpu.VMEM((N, C), jnp.float32),
        ],
        compiler_params=pltpu.CompilerParams(
            dimension_semantics=("arbitrary",),
        ),
    )(Adj_, x, W1, b1r, W2, b2r)

    return (out, adj_out)
